# baseline (speedup 1.0000x reference)
import jax
import jax.numpy as jnp
from jax.experimental import pallas as pl

BM = 64


def _fused_body(p_ref, x_ref, w1_ref, b1_ref, w2_ref, b2_ref, out_ref):
    pooled = jnp.dot(p_ref[...], x_ref[...],
                     preferred_element_type=jnp.float32)
    h = jnp.dot(pooled, w1_ref[...], preferred_element_type=jnp.float32)
    h = h + b1_ref[...]
    h = 0.5 * h * (1.0 + jax.lax.erf(h * 0.7071067811865476))
    out = jnp.dot(h, w2_ref[...], preferred_element_type=jnp.float32)
    out_ref[...] = out + b2_ref[...]


def kernel(x, W1, b1, W2, b2):
    n, c, d = x.shape
    h_dim = W1.shape[1]
    q = 256
    win = c // q
    m = n * q
    xf = x.reshape(m * win, d)
    rows = jax.lax.broadcasted_iota(jnp.int32, (BM, win * BM), 0)
    cols = jax.lax.broadcasted_iota(jnp.int32, (BM, win * BM), 1)
    pool_mat = jnp.where(cols // win == rows, 1.0 / win, 0.0).astype(jnp.float32)
    grid = (m // BM,)
    out = pl.pallas_call(
        _fused_body,
        grid=grid,
        in_specs=[
            pl.BlockSpec((BM, win * BM), lambda i: (0, 0)),
            pl.BlockSpec((BM * win, d), lambda i: (i, 0)),
            pl.BlockSpec((d, h_dim), lambda i: (0, 0)),
            pl.BlockSpec((1, h_dim), lambda i: (0, 0)),
            pl.BlockSpec((h_dim, d), lambda i: (0, 0)),
            pl.BlockSpec((1, d), lambda i: (0, 0)),
        ],
        out_specs=pl.BlockSpec((BM, d), lambda i: (i, 0)),
        out_shape=jax.ShapeDtypeStruct((m, d), jnp.float32),
    )(pool_mat, xf, W1, b1.reshape(1, h_dim), W2, b2.reshape(1, d))
    return out


# speedup vs baseline: 1.3459x; 1.3459x over previous
import jax
import jax.numpy as jnp
from jax.experimental import pallas as pl
from jax.experimental.pallas import tpu as pltpu

BM = 128


def _make_body(nb, win):
    slab = BM * win

    def _body(p_ref, b1_ref, b2_ref, x_hbm, w1_hbm, w2_hbm, out_hbm,
              xbuf, w1v, w2v, obuf, xsem, w1sem, w2sem, osem):
        def xcopy(i):
            return pltpu.make_async_copy(
                x_hbm.at[pl.ds(i * slab, slab), :], xbuf.at[i % 2],
                xsem.at[i % 2])

        def ocopy(i):
            return pltpu.make_async_copy(
                obuf.at[i % 2], out_hbm.at[pl.ds(i * BM, BM), :],
                osem.at[i % 2])

        cp1 = pltpu.make_async_copy(w1_hbm, w1v, w1sem)
        cp2 = pltpu.make_async_copy(w2_hbm, w2v, w2sem)
        xcopy(0).start()
        cp1.start()
        xcopy(1).start()
        cp2.start()

        for i in range(nb):
            xcopy(i).wait()
            xb = xbuf[i % 2]
            resid = xb - xb.astype(jnp.bfloat16).astype(jnp.float32)
            pooled = (jnp.dot(p_ref[...], xb,
                              preferred_element_type=jnp.float32)
                      + jnp.dot(p_ref[...], resid,
                                preferred_element_type=jnp.float32))
            if i + 2 < nb:
                xcopy(i + 2).start()
            if i == 0:
                cp1.wait()
            h = jnp.dot(pooled, w1v[...], preferred_element_type=jnp.float32)
            h = h + b1_ref[...]
            h = 0.5 * h * (1.0 + jax.lax.erf(h * 0.7071067811865476))
            if i == 0:
                cp2.wait()
            if i >= 2:
                ocopy(i - 2).wait()
            obuf[i % 2] = jnp.dot(
                h, w2v[...], preferred_element_type=jnp.float32) + b2_ref[...]
            ocopy(i).start()

        ocopy(nb - 2).wait()
        ocopy(nb - 1).wait()

    return _body


def kernel(x, W1, b1, W2, b2):
    n, c, d = x.shape
    h_dim = W1.shape[1]
    q = 256
    win = c // q
    m = n * q
    nb = m // BM
    xf = x.reshape(m * win, d)
    rows = jax.lax.broadcasted_iota(jnp.int32, (BM, win * BM), 0)
    cols = jax.lax.broadcasted_iota(jnp.int32, (BM, win * BM), 1)
    pool_mat = jnp.where(cols // win == rows, 1.0 / win, 0.0).astype(jnp.float32)
    vmem = pl.BlockSpec(memory_space=pltpu.MemorySpace.VMEM)
    hbm = pl.BlockSpec(memory_space=pltpu.MemorySpace.HBM)
    out = pl.pallas_call(
        _make_body(nb, win),
        in_specs=[vmem, vmem, vmem, hbm, hbm, hbm],
        out_specs=hbm,
        out_shape=jax.ShapeDtypeStruct((m, d), jnp.float32),
        scratch_shapes=[
            pltpu.VMEM((2, BM * win, d), jnp.float32),
            pltpu.VMEM((d, h_dim), jnp.float32),
            pltpu.VMEM((h_dim, d), jnp.float32),
            pltpu.VMEM((2, BM, d), jnp.float32),
            pltpu.SemaphoreType.DMA((2,)),
            pltpu.SemaphoreType.DMA,
            pltpu.SemaphoreType.DMA,
            pltpu.SemaphoreType.DMA((2,)),
        ],
    )(pool_mat, b1.reshape(1, h_dim), b2.reshape(1, d), xf, W1, W2)
    return out


# speedup vs baseline: 1.5105x; 1.1223x over previous
import jax
import jax.numpy as jnp
from jax.experimental import pallas as pl
from jax.experimental.pallas import tpu as pltpu

BM = 128


def _make_body(nb, win):
    slab = BM * win

    def _body(p_ref, b1_ref, b2_ref, x_hbm, w1_hbm, w2_hbm, out_hbm,
              xbuf, w1v, w2v, obuf, xsem, w1sem, w2sem, osem):
        def xcopy(i):
            return pltpu.make_async_copy(
                x_hbm.at[pl.ds(i * slab, slab), :], xbuf.at[i % 2],
                xsem.at[i % 2])

        def ocopy(i):
            return pltpu.make_async_copy(
                obuf.at[i % 2], out_hbm.at[pl.ds(i * BM, BM), :],
                osem.at[i % 2])

        cp1 = pltpu.make_async_copy(w1_hbm, w1v, w1sem)
        cp2 = pltpu.make_async_copy(w2_hbm, w2v, w2sem)
        xcopy(0).start()
        cp1.start()
        xcopy(1).start()
        cp2.start()

        for i in range(nb):
            xcopy(i).wait()
            pooled = jnp.dot(p_ref[...], xbuf[i % 2],
                             preferred_element_type=jnp.float32)
            if i + 2 < nb:
                xcopy(i + 2).start()
            if i == 0:
                cp1.wait()
            h = jnp.dot(pooled, w1v[...], preferred_element_type=jnp.float32)
            h = h + b1_ref[...]
            h = 0.5 * h * (1.0 + jax.lax.erf(h * 0.7071067811865476))
            if i == 0:
                cp2.wait()
            if i >= 2:
                ocopy(i - 2).wait()
            obuf[i % 2] = jnp.dot(
                h, w2v[...], preferred_element_type=jnp.float32) + b2_ref[...]
            ocopy(i).start()

        ocopy(nb - 2).wait()
        ocopy(nb - 1).wait()

    return _body


def kernel(x, W1, b1, W2, b2):
    n, c, d = x.shape
    h_dim = W1.shape[1]
    q = 256
    win = c // q
    m = n * q
    nb = m // BM
    xf = x.reshape(m * win, d)
    rows = jax.lax.broadcasted_iota(jnp.int32, (BM, win * BM), 0)
    cols = jax.lax.broadcasted_iota(jnp.int32, (BM, win * BM), 1)
    pool_mat = jnp.where(cols // win == rows, 1.0 / win, 0.0).astype(jnp.float32)
    vmem = pl.BlockSpec(memory_space=pltpu.MemorySpace.VMEM)
    hbm = pl.BlockSpec(memory_space=pltpu.MemorySpace.HBM)
    out = pl.pallas_call(
        _make_body(nb, win),
        in_specs=[vmem, vmem, vmem, hbm, hbm, hbm],
        out_specs=hbm,
        out_shape=jax.ShapeDtypeStruct((m, d), jnp.float32),
        scratch_shapes=[
            pltpu.VMEM((2, BM * win, d), jnp.float32),
            pltpu.VMEM((d, h_dim), jnp.float32),
            pltpu.VMEM((h_dim, d), jnp.float32),
            pltpu.VMEM((2, BM, d), jnp.float32),
            pltpu.SemaphoreType.DMA((2,)),
            pltpu.SemaphoreType.DMA,
            pltpu.SemaphoreType.DMA,
            pltpu.SemaphoreType.DMA((2,)),
        ],
    )(pool_mat, b1.reshape(1, h_dim), b2.reshape(1, d), xf, W1, W2)
    return out
